# baseline (speedup 1.0000x reference)
import jax
import jax.numpy as jnp
from jax.experimental import pallas as pl
from jax.experimental.pallas import tpu as pltpu


def _moe_body(x_ref, g_ref, w_ref, b_ref, o_ref):
    x = x_ref[...].astype(jnp.bfloat16)
    g = jnp.maximum(g_ref[...], 0.0)
    gb = g.astype(jnp.bfloat16)
    n = b_ref.shape[0]
    xs = jnp.concatenate(
        [gb[:, i:i + 1] * x for i in range(n)], axis=1)
    acc = jax.lax.dot_general(
        gb, b_ref[...],
        (((1,), (0,)), ((), ())),
        preferred_element_type=jnp.float32,
    )
    acc = acc + jax.lax.dot_general(
        xs, w_ref[...],
        (((1,), (0,)), ((), ())),
        preferred_element_type=jnp.float32,
    )
    o_ref[...] = acc


def kernel(input, gates, W, b):
    in_shape = input.shape
    d_in = in_shape[-1]
    n = gates.shape[-1]
    d_out = W.shape[-1]
    x = jnp.reshape(input, (-1, d_in))
    g = jnp.reshape(gates, (-1, n))
    t = x.shape[0]

    for tb in (1024, 512, 256, 128, 64, 32, 16, 8):
        if t % tb == 0:
            break
    else:
        tb = t
    grid = (t // tb,)

    w_stack = jnp.reshape(W, (n * d_in, d_out)).astype(jnp.bfloat16)
    b_bf16 = b.astype(jnp.bfloat16)

    out = pl.pallas_call(
        _moe_body,
        grid=grid,
        in_specs=[
            pl.BlockSpec((tb, d_in), lambda i: (i, 0)),
            pl.BlockSpec((tb, n), lambda i: (i, 0)),
            pl.BlockSpec((n * d_in, d_out), lambda i: (0, 0)),
            pl.BlockSpec((n, d_out), lambda i: (0, 0)),
        ],
        out_specs=pl.BlockSpec((tb, d_out), lambda i: (i, 0)),
        out_shape=jax.ShapeDtypeStruct((t, d_out), jnp.float32),
        compiler_params=pltpu.CompilerParams(
            dimension_semantics=("parallel",),
            allow_input_fusion=[False, False, True, True],
        ),
    )(x, g, w_stack, b_bf16)

    return jnp.reshape(out, tuple(in_shape[:-1]) + (d_out,))
